# baseline (speedup 1.0000x reference)
import functools

import jax
import jax.numpy as jnp
from jax import lax
from jax.experimental import pallas as pl
from jax.experimental.pallas import tpu as pltpu
from jax.experimental.pallas import tpu_sc as plsc

HIDDEN = 512
B_TOTAL = 4096 * 200
NC, NS = 2, 16
NW = NC * NS
B_PER_W = B_TOTAL // NW
CB = 120
N_FULL = B_PER_W // CB
TAIL = B_PER_W - N_FULL * CB
NBUF = 2

assert CB % 8 == 0 and CB <= 128 and TAIL % 8 == 0
assert N_FULL % 2 == 1
assert NBUF * CB * (HIDDEN + 1) * 4 <= 524284


def _emb_body(table_hbm, idx_hbm, out_hbm, idx_v, rows_v, sem0, sem1):
    sems = (sem0, sem1)
    wid = lax.axis_index("s") * NC + lax.axis_index("c")
    base = wid * B_PER_W

    def load_idx(g, b, n=CB):
        pltpu.sync_copy(idx_hbm.at[pl.ds(base + g * CB, n)],
                        idx_v.at[b, pl.ds(0, n)])

    def start_gather(b, n=CB):
        pltpu.async_copy(table_hbm.at[idx_v.at[b, pl.ds(0, n)]],
                         rows_v.at[b, pl.ds(0, n)], sems[b])

    def wait_gather(b, n=CB):
        pltpu.make_async_copy(table_hbm.at[idx_v.at[b, pl.ds(0, n)]],
                              rows_v.at[b, pl.ds(0, n)], sems[b]).wait()

    def store(g, b, n=CB):
        pltpu.sync_copy(rows_v.at[b, pl.ds(0, n)],
                        out_hbm.at[pl.ds(base + g * CB, n)])

    load_idx(0, 0)
    start_gather(0)

    def blk_body(blk, carry):
        for b in range(NBUF):
            g = blk * NBUF + b
            load_idx(g + 1, 1 - b)
            start_gather(1 - b)
            wait_gather(b)
            store(g, b)
        return carry

    lax.fori_loop(0, (N_FULL - 1) // 2, blk_body, 0)

    g0 = N_FULL - 1
    load_idx(g0 + 1, 1, TAIL)
    start_gather(1, TAIL)
    wait_gather(0)
    store(g0, 0)
    wait_gather(1, TAIL)
    store(g0 + 1, 1, TAIL)


_gather_call = functools.partial(
    pl.kernel,
    out_type=jax.ShapeDtypeStruct((B_TOTAL, HIDDEN), jnp.float32),
    mesh=plsc.VectorSubcoreMesh(core_axis_name="c", subcore_axis_name="s"),
    scratch_types=[
        pltpu.VMEM((NBUF, CB), jnp.int32),
        pltpu.VMEM((NBUF, CB, HIDDEN), jnp.float32),
        pltpu.SemaphoreType.DMA,
        pltpu.SemaphoreType.DMA,
    ],
)(_emb_body)


def kernel(text, embedding_table):
    flat_idx = text.reshape(-1).astype(jnp.int32)
    out = _gather_call(embedding_table, flat_idx)
    return out.reshape(text.shape + (embedding_table.shape[-1],))
